# baseline (speedup 1.0000x reference)
import functools

import jax
import jax.numpy as jnp
from jax import lax
from jax.experimental import pallas as pl
from jax.experimental.pallas import tpu as pltpu
from jax.experimental.pallas import tpu_sc as plsc

NUM_CORES = 2
NUM_SUBCORES = 16
LANES = 16
NUM_WORKERS = NUM_CORES * NUM_SUBCORES

BATCH = 16384
EMBED_DIM = 32
GLOBAL_MEAN = 3.5
ROWS_PER_WORKER = BATCH // NUM_WORKERS
CHUNKS = ROWS_PER_WORKER // LANES


def _svd_body(num_rows, x_hbm, p_hbm, q_hbm, bu_hbm, bi_hbm, out_hbm,
              uidx_v, iidx_v, prows_v, qrows_v, bu_v, bi_v, out_v, sem):
    wid = lax.axis_index("s") * NUM_CORES + lax.axis_index("c")
    base = wid * ROWS_PER_WORKER

    pltpu.sync_copy(x_hbm.at[0, pl.ds(base, ROWS_PER_WORKER)], uidx_v)
    pltpu.sync_copy(x_hbm.at[1, pl.ds(base, ROWS_PER_WORKER)], iidx_v)

    def fix_indices(k, _):
        sl = pl.ds(k * LANES, LANES)
        u = uidx_v[sl] - 1
        uidx_v[sl] = jnp.where(u < 0, u + num_rows, u)
        i = iidx_v[sl] - 1
        iidx_v[sl] = jnp.where(i < 0, i + num_rows, i)
        return _

    lax.fori_loop(0, CHUNKS, fix_indices, 0, unroll=4)

    cp = pltpu.async_copy(p_hbm.at[uidx_v], prows_v, sem)
    cq = pltpu.async_copy(q_hbm.at[iidx_v], qrows_v, sem)
    cu = pltpu.async_copy(bu_hbm.at[uidx_v], bu_v, sem)
    ci = pltpu.async_copy(bi_hbm.at[iidx_v], bi_v, sem)
    cp.wait()
    cq.wait()
    cu.wait()
    ci.wait()

    lane = lax.iota(jnp.int32, LANES)

    def chunk(k, _):
        sl = pl.ds(k * LANES, LANES)
        acc = bu_v[sl] + bi_v[sl] + GLOBAL_MEAN

        for j in range(LANES):
            r = k * LANES + j
            prow = prows_v[r, :]
            qrow = qrows_v[r, :]
            pa, pb = plsc.unpack(prow, format=plsc.PackFormat.INTERLEAVED)
            qa, qb = plsc.unpack(qrow, format=plsc.PackFormat.INTERLEAVED)
            prod = pa * qa + pb * qb
            tot = jnp.sum(prod)
            acc = jnp.where(lane == j, acc + tot, acc)
        out_v[sl] = acc
        return _

    lax.fori_loop(0, CHUNKS, chunk, 0)

    pltpu.sync_copy(out_v, out_hbm.at[pl.ds(base, ROWS_PER_WORKER)])


def kernel(x, P, Q, B_U, B_I):
    num_rows = P.shape[0]
    pb = P.astype(jnp.bfloat16)
    qb = Q.astype(jnp.bfloat16)
    bu_flat = B_U.reshape(num_rows)
    bi_flat = B_I.reshape(num_rows)
    mesh = plsc.VectorSubcoreMesh(core_axis_name="c", subcore_axis_name="s")
    call = pl.kernel(
        functools.partial(_svd_body, num_rows),
        out_type=jax.ShapeDtypeStruct((BATCH,), jnp.float32),
        mesh=mesh,
        scratch_types=[
            pltpu.VMEM((ROWS_PER_WORKER,), jnp.int32),
            pltpu.VMEM((ROWS_PER_WORKER,), jnp.int32),
            pltpu.VMEM((ROWS_PER_WORKER, EMBED_DIM), jnp.bfloat16),
            pltpu.VMEM((ROWS_PER_WORKER, EMBED_DIM), jnp.bfloat16),
            pltpu.VMEM((ROWS_PER_WORKER,), jnp.float32),
            pltpu.VMEM((ROWS_PER_WORKER,), jnp.float32),
            pltpu.VMEM((ROWS_PER_WORKER,), jnp.float32),
            pltpu.SemaphoreType.DMA,
        ],
        compiler_params=pltpu.CompilerParams(
            needs_layout_passes=False, use_tc_tiling_on_sc=False
        ),
    )
    return call(x, pb, qb, bu_flat, bi_flat)


# speedup vs baseline: 3.8227x; 3.8227x over previous
import functools

import jax
import jax.numpy as jnp
from jax import lax
from jax.experimental import pallas as pl
from jax.experimental.pallas import tpu as pltpu
from jax.experimental.pallas import tpu_sc as plsc

NUM_CORES = 2
NUM_SUBCORES = 16
LANES = 16
NUM_WORKERS = NUM_CORES * NUM_SUBCORES

BATCH = 16384
EMBED_DIM = 32
TILE_W = 128
GLOBAL_MEAN = 3.5
ROWS_PER_WORKER = BATCH // NUM_WORKERS
WAVE = 4
NBUF = 2
WAVES_PER_CHUNK = LANES // WAVE
NWAVES = ROWS_PER_WORKER // WAVE
NCHUNKS = ROWS_PER_WORKER // LANES
BLOCK_BYTES = EMBED_DIM * TILE_W * 4
WAVE_BYTES = 2 * WAVE * BLOCK_BYTES


def _svd_body(num_rows, x_hbm, pt_hbm, qt_hbm, bu_hbm, bi_hbm, out_hbm,
              uidx_v, iidx_v, pblk_v, qblk_v,
              bu_v, bi_v, out_v, gsem, sem):
    wid = lax.axis_index("s") * NUM_CORES + lax.axis_index("c")
    base = wid * ROWS_PER_WORKER

    pltpu.sync_copy(
        x_hbm.at[0, pl.ds(base, ROWS_PER_WORKER)],
        uidx_v.at[pl.ds(0, ROWS_PER_WORKER)],
    )
    pltpu.sync_copy(
        x_hbm.at[1, pl.ds(base, ROWS_PER_WORKER)],
        iidx_v.at[pl.ds(0, ROWS_PER_WORKER)],
    )

    def fix_indices(k, _):
        sl = pl.ds(k * LANES, LANES)
        u = uidx_v[sl] - 1
        uidx_v[sl] = jnp.where(u < 0, u + num_rows, u)
        i = iidx_v[sl] - 1
        iidx_v[sl] = jnp.where(i < 0, i + num_rows, i)
        return _

    lax.fori_loop(0, NCHUNKS, fix_indices, 0, unroll=4)

    cu = pltpu.async_copy(
        bu_hbm.at[uidx_v.at[pl.ds(0, ROWS_PER_WORKER)]], bu_v, gsem
    )
    ci = pltpu.async_copy(
        bi_hbm.at[iidx_v.at[pl.ds(0, ROWS_PER_WORKER)]], bi_v, gsem
    )

    cu.wait()
    ci.wait()

    def fixed_ids(k):
        uv = uidx_v[pl.ds(k, LANES)]
        iv = iidx_v[pl.ds(k, LANES)]
        return uv[0], iv[0]

    lane = lax.iota(jnp.int32, LANES)
    dlo = lane
    dhi = lane + LANES

    def fetch_wave(w, buf):
        for j in range(WAVE):
            k = jnp.minimum(w * WAVE + j, ROWS_PER_WORKER - 1)
            u, i = fixed_ids(k)
            ub = pl.multiple_of((u // TILE_W) * TILE_W, TILE_W)
            ib = pl.multiple_of((i // TILE_W) * TILE_W, TILE_W)
            pltpu.async_copy(
                pt_hbm.at[:, pl.ds(ub, TILE_W)], pblk_v.at[buf, j], sem
            )
            pltpu.async_copy(
                qt_hbm.at[:, pl.ds(ib, TILE_W)], qblk_v.at[buf, j], sem
            )

    def drain_wave():
        for _ in range(2 * WAVE):
            pltpu.make_async_copy(
                pt_hbm.at[:, pl.ds(0, TILE_W)], pblk_v.at[0, 0], sem
            ).wait()

    def compute_wave(w, m, buf, acc):
        for j in range(WAVE):
            k = w * WAVE + j
            u, i = fixed_ids(k)
            uoff = jnp.broadcast_to(u % TILE_W, (LANES,))
            ioff = jnp.broadcast_to(i % TILE_W, (LANES,))
            p1 = plsc.load_gather(pblk_v.at[buf, j], [dlo, uoff])
            p2 = plsc.load_gather(pblk_v.at[buf, j], [dhi, uoff])
            q1 = plsc.load_gather(qblk_v.at[buf, j], [dlo, ioff])
            q2 = plsc.load_gather(qblk_v.at[buf, j], [dhi, ioff])
            tot = jnp.sum(p1 * q1 + p2 * q2)
            acc = jnp.where(lane == (m * WAVE + j), acc + tot, acc)
        return acc

    fetch_wave(jnp.int32(0), 0)
    fetch_wave(jnp.int32(1), 1)

    def chunk_body(g, _):
        acc = jnp.zeros((LANES,), jnp.float32)
        for m in range(WAVES_PER_CHUNK):
            w = g * WAVES_PER_CHUNK + m
            buf = m % NBUF
            drain_wave()
            acc = compute_wave(w, m, buf, acc)
            fetch_wave(w + NBUF, buf)
        sl = pl.ds(g * LANES, LANES)
        out_v[sl] = acc + bu_v[sl] + bi_v[sl] + GLOBAL_MEAN
        return _

    lax.fori_loop(0, NCHUNKS, chunk_body, 0)

    drain_wave()
    drain_wave()

    pltpu.sync_copy(out_v, out_hbm.at[pl.ds(base, ROWS_PER_WORKER)])


def kernel(x, P, Q, B_U, B_I):
    num_rows = P.shape[0]
    pt = P.T
    qt = Q.T
    bu_flat = B_U.reshape(num_rows)
    bi_flat = B_I.reshape(num_rows)
    mesh = plsc.VectorSubcoreMesh(core_axis_name="c", subcore_axis_name="s")
    call = pl.kernel(
        functools.partial(_svd_body, num_rows),
        out_type=jax.ShapeDtypeStruct((BATCH,), jnp.float32),
        mesh=mesh,
        scratch_types=[
            pltpu.VMEM((ROWS_PER_WORKER + LANES,), jnp.int32),
            pltpu.VMEM((ROWS_PER_WORKER + LANES,), jnp.int32),
            pltpu.VMEM((NBUF, WAVE, EMBED_DIM, TILE_W), jnp.float32),
            pltpu.VMEM((NBUF, WAVE, EMBED_DIM, TILE_W), jnp.float32),
            pltpu.VMEM((ROWS_PER_WORKER,), jnp.float32),
            pltpu.VMEM((ROWS_PER_WORKER,), jnp.float32),
            pltpu.VMEM((ROWS_PER_WORKER,), jnp.float32),
            pltpu.SemaphoreType.DMA,
            pltpu.SemaphoreType.DMA,
        ],
        compiler_params=pltpu.CompilerParams(needs_layout_passes=False),
    )
    return call(x, pt, qt, bu_flat, bi_flat)


# speedup vs baseline: 3.8228x; 1.0000x over previous
import functools

import jax
import jax.numpy as jnp
from jax import lax
from jax.experimental import pallas as pl
from jax.experimental.pallas import tpu as pltpu
from jax.experimental.pallas import tpu_sc as plsc

NUM_CORES = 2
NUM_SUBCORES = 16
LANES = 16
NUM_WORKERS = NUM_CORES * NUM_SUBCORES

BATCH = 16384
EMBED_DIM = 32
TILE_W = 128
GLOBAL_MEAN = 3.5
ROWS_PER_WORKER = BATCH // NUM_WORKERS
WAVE = 4
NBUF = 2
WAVES_PER_CHUNK = LANES // WAVE
NCHUNKS = ROWS_PER_WORKER // LANES


def _svd_body(num_rows, x_hbm, pt_hbm, qt_hbm, bu_hbm, bi_hbm, out_hbm,
              uidx_v, iidx_v, pblk_v, qblk_v,
              bu_v, bi_v, out_v, gsem, sem):
    wid = lax.axis_index("s") * NUM_CORES + lax.axis_index("c")
    base = wid * ROWS_PER_WORKER

    pltpu.sync_copy(
        x_hbm.at[0, pl.ds(base, ROWS_PER_WORKER)],
        uidx_v.at[pl.ds(0, ROWS_PER_WORKER)],
    )
    pltpu.sync_copy(
        x_hbm.at[1, pl.ds(base, ROWS_PER_WORKER)],
        iidx_v.at[pl.ds(0, ROWS_PER_WORKER)],
    )

    def fix_indices(k, _):
        sl = pl.ds(k * LANES, LANES)
        u = uidx_v[sl] - 1
        uidx_v[sl] = jnp.where(u < 0, u + num_rows, u)
        i = iidx_v[sl] - 1
        iidx_v[sl] = jnp.where(i < 0, i + num_rows, i)
        return _

    lax.fori_loop(0, NCHUNKS, fix_indices, 0, unroll=4)

    cu = pltpu.async_copy(
        bu_hbm.at[uidx_v.at[pl.ds(0, ROWS_PER_WORKER)]], bu_v, gsem
    )
    ci = pltpu.async_copy(
        bi_hbm.at[iidx_v.at[pl.ds(0, ROWS_PER_WORKER)]], bi_v, gsem
    )

    cu.wait()
    ci.wait()

    def fixed_ids(k):
        uv = uidx_v[pl.ds(k, LANES)]
        iv = iidx_v[pl.ds(k, LANES)]
        return uv[0], iv[0]

    lane = lax.iota(jnp.int32, LANES)
    dlo = lane
    dhi = lane + LANES

    def fetch_wave(w, buf):
        for j in range(WAVE):
            k = jnp.minimum(w * WAVE + j, ROWS_PER_WORKER - 1)
            u, i = fixed_ids(k)
            ub = pl.multiple_of((u // TILE_W) * TILE_W, TILE_W)
            ib = pl.multiple_of((i // TILE_W) * TILE_W, TILE_W)
            pltpu.async_copy(
                pt_hbm.at[:, pl.ds(ub, TILE_W)], pblk_v.at[buf, j], sem
            )
            pltpu.async_copy(
                qt_hbm.at[:, pl.ds(ib, TILE_W)], qblk_v.at[buf, j], sem
            )

    def drain_wave():
        for _ in range(2 * WAVE):
            pltpu.make_async_copy(
                pt_hbm.at[:, pl.ds(0, TILE_W)], pblk_v.at[0, 0], sem
            ).wait()

    def compute_wave(w, m, buf, acc):
        for j in range(WAVE):
            k = w * WAVE + j
            u, i = fixed_ids(k)
            uoff = jnp.broadcast_to(u % TILE_W, (LANES,))
            ioff = jnp.broadcast_to(i % TILE_W, (LANES,))
            p1 = plsc.load_gather(pblk_v.at[buf, j], [dlo, uoff])
            p2 = plsc.load_gather(pblk_v.at[buf, j], [dhi, uoff])
            q1 = plsc.load_gather(qblk_v.at[buf, j], [dlo, ioff])
            q2 = plsc.load_gather(qblk_v.at[buf, j], [dhi, ioff])
            tot = jnp.sum(p1 * q1 + p2 * q2)
            acc = jnp.where(lane == (m * WAVE + j), acc + tot, acc)
        return acc

    fetch_wave(jnp.int32(0), 0)
    fetch_wave(jnp.int32(1), 1)

    def chunk_body(g, _):
        acc = jnp.zeros((LANES,), jnp.float32)
        for m in range(WAVES_PER_CHUNK):
            w = g * WAVES_PER_CHUNK + m
            buf = m % NBUF
            drain_wave()
            acc = compute_wave(w, m, buf, acc)
            fetch_wave(w + NBUF, buf)
        sl = pl.ds(g * LANES, LANES)
        out_v[sl] = acc + bu_v[sl] + bi_v[sl] + GLOBAL_MEAN
        return _

    lax.fori_loop(0, NCHUNKS, chunk_body, 0)

    drain_wave()
    drain_wave()

    pltpu.sync_copy(out_v, out_hbm.at[pl.ds(base, ROWS_PER_WORKER)])


def kernel(x, P, Q, B_U, B_I):
    num_rows = P.shape[0]
    pt = P.T
    qt = Q.T
    bu_flat = B_U.reshape(num_rows)
    bi_flat = B_I.reshape(num_rows)
    mesh = plsc.VectorSubcoreMesh(core_axis_name="c", subcore_axis_name="s")
    call = pl.kernel(
        functools.partial(_svd_body, num_rows),
        out_type=jax.ShapeDtypeStruct((BATCH,), jnp.float32),
        mesh=mesh,
        scratch_types=[
            pltpu.VMEM((ROWS_PER_WORKER + LANES,), jnp.int32),
            pltpu.VMEM((ROWS_PER_WORKER + LANES,), jnp.int32),
            pltpu.VMEM((NBUF, WAVE, EMBED_DIM, TILE_W), jnp.float32),
            pltpu.VMEM((NBUF, WAVE, EMBED_DIM, TILE_W), jnp.float32),
            pltpu.VMEM((ROWS_PER_WORKER,), jnp.float32),
            pltpu.VMEM((ROWS_PER_WORKER,), jnp.float32),
            pltpu.VMEM((ROWS_PER_WORKER,), jnp.float32),
            pltpu.SemaphoreType.DMA,
            pltpu.SemaphoreType.DMA,
        ],
        compiler_params=pltpu.CompilerParams(needs_layout_passes=False),
    )
    return call(x, pt, qt, bu_flat, bi_flat)
